# baseline (speedup 1.0000x reference)
import functools

import jax
import jax.numpy as jnp
from jax import lax
from jax.experimental import pallas as pl
from jax.experimental.pallas import tpu as pltpu
from jax.experimental.pallas import tpu_sc as plsc

_NCORES = 2
_NSUB = 16
_NW = _NCORES * _NSUB
_CH = 128


def _sc_chunks_per_worker(e):
    nch = -(-e // (_CH * _NW))
    return -(-nch // 4) * 4


def _make_sc_segsum(n, h, e):
    nch = _sc_chunks_per_worker(e)
    rps = ((n + _NSUB - 1) // _NSUB + 7) // 8 * 8
    rlast = n - (_NSUB - 1) * rps
    assert rlast > 0 and n % 8 == 0

    mesh = plsc.VectorSubcoreMesh(core_axis_name="c", subcore_axis_name="s")

    scratch = [
        pltpu.VMEM_SHARED((n + 8, h), jnp.float32),
        pltpu.VMEM((_CH,), jnp.int32),
        pltpu.VMEM((_CH,), jnp.int32),
        pltpu.VMEM((_CH,), jnp.int32),
        pltpu.VMEM((_CH,), jnp.int32),
        pltpu.VMEM((_CH, h), jnp.float32),
        pltpu.VMEM((_CH, h), jnp.float32),
        pltpu.SemaphoreType.DMA,
        pltpu.SemaphoreType.DMA,
        pltpu.SemaphoreType.DMA,
        pltpu.SemaphoreType.DMA,
    ]

    @functools.partial(
        pl.kernel,
        out_type=jax.ShapeDtypeStruct((_NCORES, n, h), jnp.float32),
        mesh=mesh,
        scratch_types=scratch,
    )
    def segsum(h_hbm, src_hbm, dst_hbm, out_hbm, accum, sidx0, didx0,
               sidx1, didx1, rows0, rows1, sem0, sem1, sem0s, sem1s):
        c = lax.axis_index("c")
        s = lax.axis_index("s")
        wid = c * _NSUB + s

        @pl.when(s < _NSUB - 1)
        def _():
            off = pl.multiple_of(s * rps, 8)
            pltpu.sync_copy(h_hbm.at[pl.ds(off, rps)],
                            accum.at[pl.ds(off, rps)])

        @pl.when(s == _NSUB - 1)
        def _():
            pltpu.sync_copy(h_hbm.at[pl.ds((_NSUB - 1) * rps, rlast)],
                            accum.at[pl.ds((_NSUB - 1) * rps, rlast)])

        plsc.subcore_barrier()

        base = wid * nch * _CH

        def chunk(k, j, sidx, didx, rows, sem_g, sem_s):
            @pl.when(k > 0)
            def _():
                pltpu.make_async_copy(rows, accum.at[didx], sem_s).wait()

            off = pl.multiple_of(base + j * _CH, 8)
            pltpu.sync_copy(src_hbm.at[pl.ds(off, _CH)], sidx)
            pltpu.sync_copy(dst_hbm.at[pl.ds(off, _CH)], didx)
            pltpu.async_copy(h_hbm.at[sidx], rows, sem_g).wait()
            pltpu.make_async_copy(rows, accum.at[didx], sem_s).start(add=True)

        def body(k, carry):
            chunk(k, 2 * k, sidx0, didx0, rows0, sem0, sem0s)
            chunk(k, 2 * k + 1, sidx1, didx1, rows1, sem1, sem1s)
            return carry

        lax.fori_loop(0, nch // 2, body, 0)
        pltpu.make_async_copy(rows0, accum.at[didx0], sem0s).wait()
        pltpu.make_async_copy(rows1, accum.at[didx1], sem1s).wait()
        plsc.subcore_barrier()

        @pl.when(s < _NSUB - 1)
        def _():
            off = pl.multiple_of(s * rps, 8)
            pltpu.sync_copy(accum.at[pl.ds(off, rps)],
                            out_hbm.at[c, pl.ds(off, rps)])

        @pl.when(s == _NSUB - 1)
        def _():
            pltpu.sync_copy(accum.at[pl.ds((_NSUB - 1) * rps, rlast)],
                            out_hbm.at[c, pl.ds((_NSUB - 1) * rps, rlast)])

    return segsum


def _bn(z, g, b):
    m = jnp.mean(z, axis=0, keepdims=True)
    v = jnp.mean((z - m) * (z - m), axis=0, keepdims=True)
    return g * (z - m) / jnp.sqrt(v + 1e-5) + b


def _gin_dense(p0_ref, p1_ref, h_ref, w1, b1, g1, bb1, w2, b2, go, bo, out):
    pooled = p0_ref[...] + p1_ref[...] - h_ref[...]
    z1 = jnp.dot(pooled, w1[...], preferred_element_type=jnp.float32) + b1[...]
    hh = jnp.maximum(_bn(z1, g1[...], bb1[...]), 0.0)
    z2 = jnp.dot(hh, w2[...], preferred_element_type=jnp.float32) + b2[...]
    out[...] = jnp.maximum(_bn(z2, go[...], bo[...]), 0.0)


def _heads(h_ref, cand_ref, mask_ref,
           aw1, ab1, aw2, ab2, aw3, ab3,
           cw1, cb1, cw2, cb2, cw3, cb3,
           pi_out, v_out, *, b, n_per, hdim, nc):
    h = h_ref[...]
    h_nodes = h.reshape(b, n_per, hdim)
    h_pooled = jnp.sum(h_nodes, axis=1) / float(n_per)
    cand = cand_ref[...]
    iota = lax.broadcasted_iota(jnp.int32, (b, nc, n_per), 2)
    onehot = (cand[:, :, None] == iota).astype(jnp.float32)
    cand_feat = lax.dot_general(
        onehot, h_nodes,
        dimension_numbers=(((2,), (1,)), ((0,), (0,))),
        preferred_element_type=jnp.float32)
    rep = jnp.broadcast_to(h_pooled[:, None, :], (b, nc, hdim))
    feat = jnp.concatenate([cand_feat, rep], axis=-1).reshape(b * nc, 2 * hdim)
    s = jnp.tanh(jnp.dot(feat, aw1[...], preferred_element_type=jnp.float32)
                 + ab1[...])
    s = jnp.tanh(jnp.dot(s, aw2[...], preferred_element_type=jnp.float32)
                 + ab2[...])
    scores = (jnp.dot(s, aw3[...], preferred_element_type=jnp.float32)
              + ab3[...]).reshape(b, nc)
    scores = jnp.where(mask_ref[...] > 0, -jnp.inf, scores)
    mx = jnp.max(scores, axis=1, keepdims=True)
    ex = jnp.exp(scores - mx)
    pi_out[...] = ex / jnp.sum(ex, axis=1, keepdims=True)
    v = jnp.tanh(jnp.dot(h_pooled, cw1[...], preferred_element_type=jnp.float32)
                 + cb1[...])
    v = jnp.tanh(jnp.dot(v, cw2[...], preferred_element_type=jnp.float32)
                 + cb2[...])
    v_out[...] = (jnp.dot(v, cw3[...], preferred_element_type=jnp.float32)
                  + cb3[...]).reshape(b)


def kernel(x, graph_pool, padded_nei, adj, candidate, mask, params):
    n, d = x.shape
    e = adj.shape[1]
    b, nc = candidate.shape
    n_per = n // b
    hdim = params['gin0']['W1'].shape[1]

    segsum = _make_sc_segsum(n, hdim, e)
    nch = _sc_chunks_per_worker(e)
    e_pad = nch * _NW * _CH
    src = jnp.concatenate([adj[0], jnp.zeros((e_pad - e,), jnp.int32)])
    dst = jnp.concatenate([adj[1], jnp.full((e_pad - e,), n, jnp.int32)])

    gin_dense = pl.pallas_call(
        _gin_dense,
        out_shape=jax.ShapeDtypeStruct((n, hdim), jnp.float32),
    )

    h = x
    for l in range(2):
        p = params['gin%d' % l]
        part = segsum(h, src, dst)
        h = gin_dense(part[0], part[1], h,
                      p['W1'], p['b1'], p['bn1_g'], p['bn1_b'],
                      p['W2'], p['b2'], p['bno_g'], p['bno_b'])

    a = params['actor']
    c = params['critic']
    heads = pl.pallas_call(
        functools.partial(_heads, b=b, n_per=n_per, hdim=hdim, nc=nc),
        out_shape=(
            jax.ShapeDtypeStruct((b, nc), jnp.float32),
            jax.ShapeDtypeStruct((b,), jnp.float32),
        ),
    )
    pi2d, v1d = heads(h, candidate, mask.astype(jnp.int32),
                      a['W1'], a['b1'], a['W2'], a['b2'], a['W3'], a['b3'],
                      c['W1'], c['b1'], c['W2'], c['b2'], c['W3'], c['b3'])
    return pi2d[:, :, None], v1d[:, None]


# speedup vs baseline: 2.6286x; 2.6286x over previous
import functools

import jax
import jax.numpy as jnp
from jax import lax
from jax.experimental import pallas as pl
from jax.experimental.pallas import tpu as pltpu
from jax.experimental.pallas import tpu_sc as plsc

_NCORES = 2
_NSUB = 16
_NW = _NCORES * _NSUB
_CH = 128


def _make_sc_segsum(n, h, e):
    ew = e // _NW
    full = ew // _CH
    tail = ew - full * _CH
    assert full % 2 == 0 and e % _NW == 0
    rps = ((n + _NSUB - 1) // _NSUB + 7) // 8 * 8
    rlast = n - (_NSUB - 1) * rps
    assert rlast > 0 and n % 8 == 0

    mesh = plsc.VectorSubcoreMesh(core_axis_name="c", subcore_axis_name="s")

    scratch = [
        pltpu.VMEM_SHARED((n, h), jnp.float32),
        pltpu.VMEM((_CH,), jnp.int32),
        pltpu.VMEM((_CH,), jnp.int32),
        pltpu.VMEM((_CH,), jnp.int32),
        pltpu.VMEM((_CH,), jnp.int32),
        pltpu.VMEM((_CH, h), jnp.float32),
        pltpu.VMEM((_CH, h), jnp.float32),
        pltpu.SemaphoreType.DMA,
        pltpu.SemaphoreType.DMA,
        pltpu.SemaphoreType.DMA,
        pltpu.SemaphoreType.DMA,
    ]
    if tail:
        scratch += [
            pltpu.VMEM((tail,), jnp.int32),
            pltpu.VMEM((tail,), jnp.int32),
            pltpu.VMEM((tail, h), jnp.float32),
        ]

    @functools.partial(
        pl.kernel,
        out_type=jax.ShapeDtypeStruct((_NCORES, n, h), jnp.float32),
        mesh=mesh,
        scratch_types=scratch,
    )
    def segsum(h_hbm, src_hbm, dst_hbm, out_hbm, accum, sidxa, didxa,
               sidxb, didxb, rowsa, rowsb, sema, semb, semsa, semsb,
               *tail_bufs):
        c = lax.axis_index("c")
        s = lax.axis_index("s")
        wid = c * _NSUB + s

        @pl.when(s < _NSUB - 1)
        def _():
            off = pl.multiple_of(s * rps, 8)
            pltpu.sync_copy(h_hbm.at[pl.ds(off, rps)],
                            accum.at[pl.ds(off, rps)])

        @pl.when(s == _NSUB - 1)
        def _():
            pltpu.sync_copy(h_hbm.at[pl.ds((_NSUB - 1) * rps, rlast)],
                            accum.at[pl.ds((_NSUB - 1) * rps, rlast)])

        plsc.subcore_barrier()

        base = wid * ew

        def body(k, carry):
            offa = pl.multiple_of(base + (2 * k) * _CH, 8)
            pltpu.sync_copy(src_hbm.at[pl.ds(offa, _CH)], sidxa)
            pltpu.sync_copy(dst_hbm.at[pl.ds(offa, _CH)], didxa)
            cpa = pltpu.async_copy(h_hbm.at[sidxa], rowsa, sema)
            offb = pl.multiple_of(base + (2 * k + 1) * _CH, 8)
            pltpu.sync_copy(src_hbm.at[pl.ds(offb, _CH)], sidxb)
            pltpu.sync_copy(dst_hbm.at[pl.ds(offb, _CH)], didxb)
            cpb = pltpu.async_copy(h_hbm.at[sidxb], rowsb, semb)
            cpa.wait()
            spa = pltpu.make_async_copy(rowsa, accum.at[didxa], semsa)
            spa.start(add=True)
            cpb.wait()
            spb = pltpu.make_async_copy(rowsb, accum.at[didxb], semsb)
            spb.start(add=True)
            spa.wait()
            spb.wait()
            return carry

        lax.fori_loop(0, full // 2, body, 0)
        if tail:
            sidx_t, didx_t, rows_t = tail_bufs
            off = pl.multiple_of(base + full * _CH, 8)
            pltpu.sync_copy(src_hbm.at[pl.ds(off, tail)], sidx_t)
            pltpu.sync_copy(dst_hbm.at[pl.ds(off, tail)], didx_t)
            pltpu.async_copy(h_hbm.at[sidx_t], rows_t, sema).wait()
            pltpu.sync_copy(rows_t, accum.at[didx_t], add=True)
        plsc.subcore_barrier()

        @pl.when(s < _NSUB - 1)
        def _():
            off = pl.multiple_of(s * rps, 8)
            pltpu.sync_copy(accum.at[pl.ds(off, rps)],
                            out_hbm.at[c, pl.ds(off, rps)])

        @pl.when(s == _NSUB - 1)
        def _():
            pltpu.sync_copy(accum.at[pl.ds((_NSUB - 1) * rps, rlast)],
                            out_hbm.at[c, pl.ds((_NSUB - 1) * rps, rlast)])

    return segsum


def _bn(z, g, b):
    m = jnp.mean(z, axis=0, keepdims=True)
    v = jnp.mean((z - m) * (z - m), axis=0, keepdims=True)
    return g * (z - m) / jnp.sqrt(v + 1e-5) + b


def _gin_dense(p0_ref, p1_ref, h_ref, w1, b1, g1, bb1, w2, b2, go, bo, out):
    pooled = p0_ref[...] + p1_ref[...] - h_ref[...]
    z1 = jnp.dot(pooled, w1[...], preferred_element_type=jnp.float32) + b1[...]
    hh = jnp.maximum(_bn(z1, g1[...], bb1[...]), 0.0)
    z2 = jnp.dot(hh, w2[...], preferred_element_type=jnp.float32) + b2[...]
    out[...] = jnp.maximum(_bn(z2, go[...], bo[...]), 0.0)


def _heads(h_ref, cand_ref, mask_ref,
           aw1, ab1, aw2, ab2, aw3, ab3,
           cw1, cb1, cw2, cb2, cw3, cb3,
           pi_out, v_out, *, b, n_per, hdim, nc):
    h = h_ref[...]
    h_nodes = h.reshape(b, n_per, hdim)
    h_pooled = jnp.sum(h_nodes, axis=1) / float(n_per)
    cand = cand_ref[...]
    iota = lax.broadcasted_iota(jnp.int32, (b, nc, n_per), 2)
    onehot = (cand[:, :, None] == iota).astype(jnp.float32)
    cand_feat = lax.dot_general(
        onehot, h_nodes,
        dimension_numbers=(((2,), (1,)), ((0,), (0,))),
        preferred_element_type=jnp.float32)
    rep = jnp.broadcast_to(h_pooled[:, None, :], (b, nc, hdim))
    feat = jnp.concatenate([cand_feat, rep], axis=-1).reshape(b * nc, 2 * hdim)
    s = jnp.tanh(jnp.dot(feat, aw1[...], preferred_element_type=jnp.float32)
                 + ab1[...])
    s = jnp.tanh(jnp.dot(s, aw2[...], preferred_element_type=jnp.float32)
                 + ab2[...])
    scores = (jnp.dot(s, aw3[...], preferred_element_type=jnp.float32)
              + ab3[...]).reshape(b, nc)
    scores = jnp.where(mask_ref[...] > 0, -jnp.inf, scores)
    mx = jnp.max(scores, axis=1, keepdims=True)
    ex = jnp.exp(scores - mx)
    pi_out[...] = ex / jnp.sum(ex, axis=1, keepdims=True)
    v = jnp.tanh(jnp.dot(h_pooled, cw1[...], preferred_element_type=jnp.float32)
                 + cb1[...])
    v = jnp.tanh(jnp.dot(v, cw2[...], preferred_element_type=jnp.float32)
                 + cb2[...])
    v_out[...] = (jnp.dot(v, cw3[...], preferred_element_type=jnp.float32)
                  + cb3[...]).reshape(b)


def kernel(x, graph_pool, padded_nei, adj, candidate, mask, params):
    n, d = x.shape
    e = adj.shape[1]
    b, nc = candidate.shape
    n_per = n // b
    hdim = params['gin0']['W1'].shape[1]

    segsum = _make_sc_segsum(n, hdim, e)
    src = adj[0]
    dst = adj[1]

    gin_dense = pl.pallas_call(
        _gin_dense,
        out_shape=jax.ShapeDtypeStruct((n, hdim), jnp.float32),
    )

    h = x
    for l in range(2):
        p = params['gin%d' % l]
        part = segsum(h, src, dst)
        h = gin_dense(part[0], part[1], h,
                      p['W1'], p['b1'], p['bn1_g'], p['bn1_b'],
                      p['W2'], p['b2'], p['bno_g'], p['bno_b'])

    a = params['actor']
    c = params['critic']
    heads = pl.pallas_call(
        functools.partial(_heads, b=b, n_per=n_per, hdim=hdim, nc=nc),
        out_shape=(
            jax.ShapeDtypeStruct((b, nc), jnp.float32),
            jax.ShapeDtypeStruct((b,), jnp.float32),
        ),
    )
    pi2d, v1d = heads(h, candidate, mask.astype(jnp.int32),
                      a['W1'], a['b1'], a['W2'], a['b2'], a['W3'], a['b3'],
                      c['W1'], c['b1'], c['W2'], c['b2'], c['W3'], c['b3'])
    return pi2d[:, :, None], v1d[:, None]
